# baseline (speedup 1.0000x reference)
import functools

import jax
import jax.numpy as jnp
from jax import lax
from jax.experimental import pallas as pl
from jax.experimental.pallas import tpu as pltpu
from jax.experimental.pallas import tpu_sc as plsc

DIM = 64
BATCH = 16384
NC = 2
NS = 16
NW = NC * NS
BPW = BATCH // NW
NCHUNK = 4
CHUNK = BPW // NCHUNK
NGROUP = BPW // 16


def _body(trip_hbm, ent_hbm, rel_hbm, out_hbm,
          trip_v, hidx, ridx, tidx, h_v, r_v, t_v, out_v, sem):
    wid = lax.axis_index("s") * NC + lax.axis_index("c")
    base = wid * BPW
    iota = lax.iota(jnp.int32, 16)

    pltpu.sync_copy(trip_hbm.at[pl.ds(base * 3, BPW * 3)], trip_v)

    for g in range(NGROUP):
        tbase = (g * 16 + iota) * 3
        row = g // (NGROUP // NCHUNK)
        col = (g % (NGROUP // NCHUNK)) * 16
        hidx[row, pl.ds(col, 16)] = plsc.load_gather(trip_v, [tbase])
        ridx[row, pl.ds(col, 16)] = plsc.load_gather(trip_v, [tbase + 1])
        tidx[row, pl.ds(col, 16)] = plsc.load_gather(trip_v, [tbase + 2])

    copies = []
    for j in range(NCHUNK):
        dst = pl.ds(j * CHUNK, CHUNK)
        copies.append(pltpu.async_copy(ent_hbm.at[hidx.at[j]], h_v.at[dst], sem))
        copies.append(pltpu.async_copy(rel_hbm.at[ridx.at[j]], r_v.at[dst], sem))
        copies.append(pltpu.async_copy(ent_hbm.at[tidx.at[j]], t_v.at[dst], sem))
    for c in copies:
        c.wait()

    def group(g, carry):
        row16 = g * 16 + iota
        accs = [jnp.zeros((16,), jnp.float32) for _ in range(4)]
        for d in range(DIM):
            cold = jnp.full((16,), d, jnp.int32)
            vh = plsc.load_gather(h_v, [row16, cold])
            vr = plsc.load_gather(r_v, [row16, cold])
            vt = plsc.load_gather(t_v, [row16, cold])
            accs[d % 4] = accs[d % 4] + jnp.abs(vh + vr - vt)
        out_v[pl.ds(g * 16, 16)] = -((accs[0] + accs[1]) + (accs[2] + accs[3]))
        return carry

    lax.fori_loop(0, NGROUP, group, jnp.int32(0))

    pltpu.sync_copy(out_v, out_hbm.at[pl.ds(base, BPW)])


@functools.partial(jax.jit, static_argnums=())
def _transe(trip_flat, entity_emb, relation_emb):
    run = functools.partial(
        pl.kernel,
        out_type=jax.ShapeDtypeStruct((BATCH,), jnp.float32),
        mesh=plsc.VectorSubcoreMesh(core_axis_name="c", subcore_axis_name="s"),
        compiler_params=pltpu.CompilerParams(
            needs_layout_passes=False, use_tc_tiling_on_sc=False),
        scratch_types=[
            pltpu.VMEM((BPW * 3,), jnp.int32),
            pltpu.VMEM((NCHUNK, CHUNK), jnp.int32),
            pltpu.VMEM((NCHUNK, CHUNK), jnp.int32),
            pltpu.VMEM((NCHUNK, CHUNK), jnp.int32),
            pltpu.VMEM((BPW, DIM), jnp.float32),
            pltpu.VMEM((BPW, DIM), jnp.float32),
            pltpu.VMEM((BPW, DIM), jnp.float32),
            pltpu.VMEM((BPW,), jnp.float32),
            pltpu.SemaphoreType.DMA,
        ],
    )(_body)
    return run(trip_flat, entity_emb, relation_emb)


def kernel(triples, entity_emb, relation_emb):
    trip_flat = triples.astype(jnp.int32).reshape(-1)
    return _transe(trip_flat, entity_emb, relation_emb)


# speedup vs baseline: 2.0478x; 2.0478x over previous
import functools

import jax
import jax.numpy as jnp
from jax import lax
from jax.experimental import pallas as pl
from jax.experimental.pallas import tpu as pltpu
from jax.experimental.pallas import tpu_sc as plsc

DIM = 64
BATCH = 16384
NC = 2
NS = 16
NW = NC * NS
BPW = BATCH // NW
CH = 32
NGROUP = BPW // 16


def _body(trip_hbm, ent_hbm, rel_hbm, out_hbm,
          trip_v, hslab, rslab, tslab, hsub, rsub, tsub,
          h_v, r_v, t_v, out_v, sem):
    wid = lax.axis_index("s") * NC + lax.axis_index("c")
    base = wid * BPW
    iota = lax.iota(jnp.int32, 16)

    pltpu.sync_copy(trip_hbm.at[pl.ds(base * 3, BPW * 3)], trip_v)

    for g in range(NGROUP):
        j16 = (g * 16 + iota) * 3
        dst = pl.ds(g * 16, 16)
        h16 = plsc.load_gather(trip_v, [j16])
        r16 = plsc.load_gather(trip_v, [j16 + 1])
        t16 = plsc.load_gather(trip_v, [j16 + 2])
        hslab[dst] = lax.shift_right_logical(h16, 3)
        rslab[dst] = lax.shift_right_logical(r16, 3)
        tslab[dst] = lax.shift_right_logical(t16, 3)
        hsub[dst] = h16 & 7
        rsub[dst] = r16 & 7
        tsub[dst] = t16 & 7

    def group(g, carry):
        c = g >> 1

        @pl.when((g & 1) == 0)
        def _dma():
            copies = []
            for gg in range(CH // 16):
                src = pl.ds(c * CH + gg * 16, 16)
                vh = hslab[src]
                vr = rslab[src]
                vt = tslab[src]
                for j in range(16):
                    slot = gg * 16 + j
                    copies.append(pltpu.async_copy(
                        ent_hbm.at[vh[j]], h_v.at[slot], sem))
                    copies.append(pltpu.async_copy(
                        rel_hbm.at[vr[j]], r_v.at[slot], sem))
                    copies.append(pltpu.async_copy(
                        ent_hbm.at[vt[j]], t_v.at[slot], sem))
            for cp in copies:
                cp.wait()

        slot16 = (g & 1) * 16 + iota
        sh = hsub[pl.ds(g * 16, 16)]
        sr = rsub[pl.ds(g * 16, 16)]
        st = tsub[pl.ds(g * 16, 16)]
        accs = [jnp.zeros((16,), jnp.float32) for _ in range(4)]
        for d in range(DIM):
            cold = jnp.full((16,), d, jnp.int32)
            vh = plsc.load_gather(h_v, [slot16, sh, cold])
            vr = plsc.load_gather(r_v, [slot16, sr, cold])
            vt = plsc.load_gather(t_v, [slot16, st, cold])
            accs[d % 4] = accs[d % 4] + jnp.abs(vh + vr - vt)
        out_v[pl.ds(g * 16, 16)] = -((accs[0] + accs[1]) + (accs[2] + accs[3]))
        return carry

    lax.fori_loop(0, NGROUP, group, jnp.int32(0))

    pltpu.sync_copy(out_v, out_hbm.at[pl.ds(base, BPW)])


@jax.jit
def _transe(trip_flat, ent3, rel3):
    run = functools.partial(
        pl.kernel,
        out_type=jax.ShapeDtypeStruct((BATCH,), jnp.float32),
        mesh=plsc.VectorSubcoreMesh(core_axis_name="c", subcore_axis_name="s"),
        compiler_params=pltpu.CompilerParams(needs_layout_passes=False),
        scratch_types=[
            pltpu.VMEM((BPW * 3,), jnp.int32),
            pltpu.VMEM((BPW,), jnp.int32),
            pltpu.VMEM((BPW,), jnp.int32),
            pltpu.VMEM((BPW,), jnp.int32),
            pltpu.VMEM((BPW,), jnp.int32),
            pltpu.VMEM((BPW,), jnp.int32),
            pltpu.VMEM((BPW,), jnp.int32),
            pltpu.VMEM((CH, 8, DIM), jnp.float32),
            pltpu.VMEM((CH, 8, DIM), jnp.float32),
            pltpu.VMEM((CH, 8, DIM), jnp.float32),
            pltpu.VMEM((BPW,), jnp.float32),
            pltpu.SemaphoreType.DMA,
        ],
    )(_body)
    return run(trip_flat, ent3, rel3)


def kernel(triples, entity_emb, relation_emb):
    trip_flat = triples.astype(jnp.int32).reshape(-1)
    ent3 = entity_emb.reshape(125000, 8, DIM)
    rel3 = relation_emb.reshape(125000, 8, DIM)
    return _transe(trip_flat, ent3, rel3)
